# baseline (speedup 1.0000x reference)
import jax
import jax.numpy as jnp
from jax.experimental import pallas as pl
from jax.experimental.pallas import tpu as pltpu

_B = 4096
_P = 1024
_Q = 1024
_E = 8
_QC = 256
_NQ = _Q // _QC
_S = _NQ * _E


def _moe_kernel(x_ref, w_ref, b_ref, rw_ref, out_ref, aux_ref,
                y0_ref, y1_ref, wgt_ref):
    s = pl.program_id(0)

    @pl.when(s == 0)
    def _router():
        logits = jnp.dot(x_ref[...], rw_ref[...],
                         preferred_element_type=jnp.float32)
        w = jax.nn.softmax(logits, axis=-1)
        wgt_ref[...] = w
        imp = jnp.mean(w, axis=0, keepdims=True)
        aux_ref[...] = jnp.float32(_E) * jnp.sum(imp * imp, keepdims=True)

    @pl.when((s < _S) & (s % 2 == 0))
    def _dot_even():
        y0_ref[...] = jnp.dot(x_ref[...], w_ref[0],
                              preferred_element_type=jnp.float32)

    @pl.when((s < _S) & (s % 2 == 1))
    def _dot_odd():
        y1_ref[...] = jnp.dot(x_ref[...], w_ref[0],
                              preferred_element_type=jnp.float32)

    def _epilogue(y_ref):
        ep = (s - 1) % _E
        w_all = wgt_ref[...]
        mask = jax.lax.broadcasted_iota(jnp.int32, (1, _E), 1) == ep
        wcol = jnp.sum(jnp.where(mask, w_all, 0.0), axis=1, keepdims=True)
        y = y_ref[...]

        @pl.when(ep == 0)
        def _first():
            out_ref[...] = jnp.dot(w_all, b_ref[...],
                                   preferred_element_type=jnp.float32) + wcol * y

        @pl.when(ep != 0)
        def _accum():
            out_ref[...] = out_ref[...] + wcol * y

    @pl.when((s >= 1) & (s % 2 == 1))
    def _epi_from_even():
        _epilogue(y0_ref)

    @pl.when((s >= 1) & (s % 2 == 0))
    def _epi_from_odd():
        _epilogue(y1_ref)


def kernel(inputs, expert_w, expert_b, router_w):
    def _qprev(s):
        return jnp.minimum((jnp.maximum(s, 1) - 1) // _E, _NQ - 1)

    out, aux = pl.pallas_call(
        _moe_kernel,
        grid=(_S + 1,),
        in_specs=[
            pl.BlockSpec((_B, _P), lambda s: (0, 0)),
            pl.BlockSpec(
                (1, _P, _QC),
                lambda s: (jnp.minimum(s, _S - 1) % _E, 0,
                           jnp.minimum(s // _E, _NQ - 1)),
            ),
            pl.BlockSpec((_E, _QC), lambda s: (0, _qprev(s))),
            pl.BlockSpec((_P, _E), lambda s: (0, 0)),
        ],
        out_specs=[
            pl.BlockSpec((_B, _QC), lambda s: (0, _qprev(s))),
            pl.BlockSpec((1, 1), lambda s: (0, 0)),
        ],
        out_shape=[
            jax.ShapeDtypeStruct((_B, _Q), jnp.float32),
            jax.ShapeDtypeStruct((1, 1), jnp.float32),
        ],
        scratch_shapes=[
            pltpu.VMEM((_B, _QC), jnp.float32),
            pltpu.VMEM((_B, _QC), jnp.float32),
            pltpu.VMEM((_B, _E), jnp.float32),
        ],
        compiler_params=pltpu.CompilerParams(
            dimension_semantics=("arbitrary",),
        ),
    )(inputs, expert_w, expert_b, router_w)
    return out, aux[0, 0]


# speedup vs baseline: 1.4395x; 1.4395x over previous
import jax
import jax.numpy as jnp
from jax.experimental import pallas as pl
from jax.experimental.pallas import tpu as pltpu

_B = 4096
_P = 1024
_Q = 1024
_E = 8


def _moe_kernel(x_ref, w_ref, b_ref, rw_ref, out_ref, aux_ref, wgt_ref):
    e = pl.program_id(0)

    @pl.when(e == 0)
    def _router():
        logits = jnp.dot(x_ref[...], rw_ref[...],
                         preferred_element_type=jnp.float32)
        w = jax.nn.softmax(logits, axis=-1)
        wgt_ref[...] = w
        imp = jnp.mean(w, axis=0, keepdims=True)
        aux_ref[...] = jnp.float32(_E) * jnp.sum(imp * imp, keepdims=True)
        out_ref[...] = jnp.dot(w, b_ref[...],
                               preferred_element_type=jnp.float32)

    w_all = wgt_ref[...]
    mask = jax.lax.broadcasted_iota(jnp.int32, (1, _E), 1) == e
    wcol = jnp.sum(jnp.where(mask, w_all, 0.0), axis=1, keepdims=True)

    y = jnp.dot(x_ref[...], w_ref[0], preferred_element_type=jnp.float32)
    out_ref[...] = out_ref[...] + wcol * y


def kernel(inputs, expert_w, expert_b, router_w):
    out, aux = pl.pallas_call(
        _moe_kernel,
        grid=(_E,),
        in_specs=[
            pl.BlockSpec((_B, _P), lambda e: (0, 0)),
            pl.BlockSpec((1, _P, _Q), lambda e: (e, 0, 0)),
            pl.BlockSpec((_E, _Q), lambda e: (0, 0)),
            pl.BlockSpec((_P, _E), lambda e: (0, 0)),
        ],
        out_specs=[
            pl.BlockSpec((_B, _Q), lambda e: (0, 0)),
            pl.BlockSpec((1, 1), lambda e: (0, 0)),
        ],
        out_shape=[
            jax.ShapeDtypeStruct((_B, _Q), jnp.float32),
            jax.ShapeDtypeStruct((1, 1), jnp.float32),
        ],
        scratch_shapes=[pltpu.VMEM((_B, _E), jnp.float32)],
        compiler_params=pltpu.CompilerParams(
            dimension_semantics=("arbitrary",),
        ),
    )(inputs, expert_w, expert_b, router_w)
    return out, aux[0, 0]
